# baseline (speedup 1.0000x reference)
import functools

import jax
import jax.numpy as jnp
from jax import lax
from jax.experimental import pallas as pl
from jax.experimental.pallas import tpu as pltpu
from jax.experimental.pallas import tpu_sc as plsc

N = 10000
D = 128
E = 320000

NC = 2
NS = 16
NW = NC * NS

K = 128
NBUF = 2
CHK = 10
C0 = 16
C1 = 0
NCHUNKS = NS * (C0 + C1)
NB = 80
CE = NB * K
EPAD = NCHUNKS * CHK * K

NPAD = 10240
RPT = NPAD // NS

_mesh = plsc.VectorSubcoreMesh(core_axis_name="c", subcore_axis_name="s")


@functools.partial(
    pl.kernel,
    out_type=jax.ShapeDtypeStruct((NC, 2, NPAD), jnp.float32),
    mesh=_mesh,
    scratch_types=[
        pltpu.VMEM((NB, K), jnp.int32),
        pltpu.VMEM((NB, K), jnp.int32),
        pltpu.VMEM((K,), jnp.float32),
        pltpu.VMEM_SHARED((NPAD,), jnp.float32),
        pltpu.VMEM_SHARED((NPAD,), jnp.float32),
    ],
)
def _deg_kernel(src_hbm, dst_hbm, zeros1_hbm, ones_hbm, out_hbm,
                sidx, didx, ones_v, acc_s, acc_d):
    c = lax.axis_index("c")
    s = lax.axis_index("s")
    wid = s * NC + c
    pltpu.sync_copy(ones_hbm, ones_v)
    pltpu.sync_copy(src_hbm.at[wid], sidx)
    pltpu.sync_copy(dst_hbm.at[wid], didx)
    pltpu.sync_copy(zeros1_hbm, acc_s.at[pl.ds(s * RPT, RPT)])
    pltpu.sync_copy(zeros1_hbm, acc_d.at[pl.ds(s * RPT, RPT)])
    plsc.subcore_barrier()

    def step(j, carry):
        pltpu.sync_copy(ones_v, acc_s.at[sidx.at[j]], add=True)
        pltpu.sync_copy(ones_v, acc_d.at[didx.at[j]], add=True)
        return carry

    lax.fori_loop(0, NB, step, 0)
    plsc.subcore_barrier()
    pltpu.sync_copy(acc_s.at[pl.ds(s * RPT, RPT)],
                    out_hbm.at[c, 0, pl.ds(s * RPT, RPT)])
    pltpu.sync_copy(acc_d.at[pl.ds(s * RPT, RPT)],
                    out_hbm.at[c, 1, pl.ds(s * RPT, RPT)])


@functools.partial(
    pl.kernel,
    out_type=jax.ShapeDtypeStruct((1, NPAD, D), jnp.float32),
    mesh=_mesh,
    scratch_types=[
        pltpu.VMEM((2, CHK, K), jnp.int32),
        pltpu.VMEM((2, CHK, K), jnp.int32),
        pltpu.VMEM((NBUF, K, D), jnp.float32),
        pltpu.VMEM_SHARED((NPAD, D), jnp.float32),
        [pltpu.SemaphoreType.DMA] * NBUF,
        pltpu.SemaphoreType.DMA,
    ],
)
def _agg_kernel(g_hbm, src_hbm, dst_hbm, zeros2_hbm, out_hbm,
                sidx, didx, rows, acc, gsems, isem):
    c = lax.axis_index("c")
    s = lax.axis_index("s")
    myc = lax.select(c == 0, C0, C1)
    cbase = lax.select(c == 0, s * C0, NS * C0 + s * C1)
    with jax.named_scope("agg_init"):
        @pl.when(c == 0)
        def _():
            pltpu.sync_copy(src_hbm.at[cbase], sidx.at[0])
            pltpu.sync_copy(dst_hbm.at[cbase], didx.at[0])
            pltpu.sync_copy(zeros2_hbm, acc.at[pl.ds(s * RPT, RPT), :])
        plsc.subcore_barrier()

    def chunk_body(ck, carry):
        par = lax.rem(ck, 2)
        nxt = lax.rem(ck + 1, 2)

        @pl.when(ck > 0)
        def _():
            pltpu.make_async_copy(src_hbm.at[cbase + ck], sidx.at[par],
                                  isem).wait()
            pltpu.make_async_copy(dst_hbm.at[cbase + ck], didx.at[par],
                                  isem).wait()

        @pl.when(ck < myc - 1)
        def _():
            pltpu.async_copy(src_hbm.at[cbase + ck + 1], sidx.at[nxt], isem)
            pltpu.async_copy(dst_hbm.at[cbase + ck + 1], didx.at[nxt], isem)

        for b in range(NBUF):
            pltpu.async_copy(g_hbm.at[sidx.at[par, b]], rows.at[b], gsems[b])
        for i in range(CHK):
            b = i % NBUF
            pltpu.make_async_copy(g_hbm.at[sidx.at[par, i]], rows.at[b],
                                  gsems[b]).wait()
            pltpu.sync_copy(rows.at[b], acc.at[didx.at[par, i]], add=True)
            if i + NBUF < CHK:
                pltpu.async_copy(g_hbm.at[sidx.at[par, i + NBUF]], rows.at[b],
                                 gsems[b])
        return carry

    with jax.named_scope("agg_loop"):
        lax.fori_loop(0, myc, chunk_body, 0)
        plsc.subcore_barrier()
    with jax.named_scope("agg_wb"):
        @pl.when(c == 0)
        def _():
            pltpu.sync_copy(acc.at[pl.ds(s * RPT, RPT), :],
                            out_hbm.at[0, pl.ds(s * RPT, RPT), :])


_BR = 400
_GRID = N // _BR

_cnt_spec = pl.BlockSpec((NC, 2, _BR, 1), lambda i: (0, 0, i, 0))
_row_spec = pl.BlockSpec((_BR, D), lambda i: (i, 0))
_w_spec = pl.BlockSpec((D, D), lambda i: (0, 0))
_b_spec = pl.BlockSpec((1, D), lambda i: (0, 0))
_p_spec = pl.BlockSpec((1, _BR, D), lambda i: (0, i, 0))


def _dot(a, b):
    return jax.lax.dot_general(a, b, (((1,), (0,)), ((), ())),
                               precision=jax.lax.Precision.HIGHEST,
                               preferred_element_type=jnp.float32)


def _stage1_body(feat_ref, w_ref, cnt_ref, o_ref):
    ns = jax.lax.rsqrt(cnt_ref[0, 0] + cnt_ref[1, 0] + 1.0)
    o_ref[...] = _dot(feat_ref[...] * ns, w_ref[...])


_stage1 = pl.pallas_call(
    _stage1_body,
    grid=(_GRID,),
    in_specs=[_row_spec, _w_spec, _cnt_spec],
    out_specs=_row_spec,
    out_shape=jax.ShapeDtypeStruct((N, D), jnp.float32),
)


def _stage_mid_body(p_ref, g_ref, cnt_ref, b_ref, w_ref, o_ref):
    nd = jax.lax.rsqrt(cnt_ref[0, 1] + cnt_ref[1, 1] + 1.0)
    ns = jax.lax.rsqrt(cnt_ref[0, 0] + cnt_ref[1, 0] + 1.0)
    agg = p_ref[0] + g_ref[...]
    h = jnp.maximum(agg * nd + b_ref[...], 0.0)
    o_ref[...] = _dot(h * ns, w_ref[...])


_stage_mid = pl.pallas_call(
    _stage_mid_body,
    grid=(_GRID,),
    in_specs=[_p_spec, _row_spec, _cnt_spec, _b_spec, _w_spec],
    out_specs=_row_spec,
    out_shape=jax.ShapeDtypeStruct((N, D), jnp.float32),
)


def _stage_final_body(p_ref, g_ref, cnt_ref, b_ref, o_ref):
    nd = jax.lax.rsqrt(cnt_ref[0, 1] + cnt_ref[1, 1] + 1.0)
    agg = p_ref[0] + g_ref[...]
    o_ref[...] = agg * nd + b_ref[...]


_stage_final = pl.pallas_call(
    _stage_final_body,
    grid=(_GRID,),
    in_specs=[_p_spec, _row_spec, _cnt_spec, _b_spec],
    out_specs=_row_spec,
    out_shape=jax.ShapeDtypeStruct((N, D), jnp.float32),
)


def kernel(feat, edge_index, W1, b1, W2, b2, W3, b3):
    pad = EPAD - E
    src = edge_index[0]
    dst = edge_index[1]
    src_a = jnp.concatenate(
        [src, jnp.zeros((pad,), jnp.int32)]).reshape(NCHUNKS, CHK, K)
    trash = N + (jnp.arange(pad, dtype=jnp.int32) % (NPAD - N))
    src_d = jnp.concatenate([src, trash]).reshape(NW, NB, K)
    dst_p = jnp.concatenate([dst, trash]).reshape(NW, NB, K)
    dst_a = dst_p.reshape(NCHUNKS, CHK, K)

    zeros1 = jnp.zeros((RPT,), jnp.float32)
    zeros2 = jnp.zeros((RPT, D), jnp.float32)
    ones = jnp.ones((K,), jnp.float32)

    cnt = _deg_kernel(src_d, dst_p, zeros1, ones)
    cnt4 = cnt.reshape(NC, 2, NPAD, 1)

    b1r = b1.reshape(1, D)
    b2r = b2.reshape(1, D)
    b3r = b3.reshape(1, D)

    g1 = _stage1(feat, W1, cnt4)
    p1 = _agg_kernel(g1, src_a, dst_a, zeros2)
    g2 = _stage_mid(p1, g1, cnt4, b1r, W2)
    p2 = _agg_kernel(g2, src_a, dst_a, zeros2)
    g3 = _stage_mid(p2, g2, cnt4, b2r, W3)
    p3 = _agg_kernel(g3, src_a, dst_a, zeros2)
    return _stage_final(p3, g3, cnt4, b3r)


# speedup vs baseline: 3.4711x; 3.4711x over previous
import functools

import jax
import jax.numpy as jnp
from jax import lax
from jax.experimental import pallas as pl
from jax.experimental.pallas import tpu as pltpu
from jax.experimental.pallas import tpu_sc as plsc

N = 10000
D = 128
E = 320000

NC = 2
NS = 16
NW = NC * NS

K = 128
NBUF = 2
CHK = 10
C0 = 8
C1 = 8
NCHUNKS = NS * (C0 + C1)
NB = 80
CE = NB * K
EPAD = NCHUNKS * CHK * K

NPAD = 10240
RPT = NPAD // NS

_mesh = plsc.VectorSubcoreMesh(core_axis_name="c", subcore_axis_name="s")


@functools.partial(
    pl.kernel,
    out_type=jax.ShapeDtypeStruct((NC, 2, NPAD), jnp.float32),
    mesh=_mesh,
    scratch_types=[
        pltpu.VMEM((NB, K), jnp.int32),
        pltpu.VMEM((NB, K), jnp.int32),
        pltpu.VMEM((K,), jnp.float32),
        pltpu.VMEM_SHARED((NPAD,), jnp.float32),
        pltpu.VMEM_SHARED((NPAD,), jnp.float32),
    ],
)
def _deg_kernel(src_hbm, dst_hbm, zeros1_hbm, ones_hbm, out_hbm,
                sidx, didx, ones_v, acc_s, acc_d):
    c = lax.axis_index("c")
    s = lax.axis_index("s")
    wid = s * NC + c
    pltpu.sync_copy(ones_hbm, ones_v)
    pltpu.sync_copy(src_hbm.at[wid], sidx)
    pltpu.sync_copy(dst_hbm.at[wid], didx)
    pltpu.sync_copy(zeros1_hbm, acc_s.at[pl.ds(s * RPT, RPT)])
    pltpu.sync_copy(zeros1_hbm, acc_d.at[pl.ds(s * RPT, RPT)])
    plsc.subcore_barrier()

    def step(j, carry):
        pltpu.sync_copy(ones_v, acc_s.at[sidx.at[j]], add=True)
        pltpu.sync_copy(ones_v, acc_d.at[didx.at[j]], add=True)
        return carry

    lax.fori_loop(0, NB, step, 0)
    plsc.subcore_barrier()
    pltpu.sync_copy(acc_s.at[pl.ds(s * RPT, RPT)],
                    out_hbm.at[c, 0, pl.ds(s * RPT, RPT)])
    pltpu.sync_copy(acc_d.at[pl.ds(s * RPT, RPT)],
                    out_hbm.at[c, 1, pl.ds(s * RPT, RPT)])


@functools.partial(
    pl.kernel,
    out_type=jax.ShapeDtypeStruct((NC, NPAD, D), jnp.float32),
    mesh=_mesh,
    scratch_types=[
        pltpu.VMEM((2, CHK, K), jnp.int32),
        pltpu.VMEM((2, CHK, K), jnp.int32),
        pltpu.VMEM((NBUF, K, D), jnp.float32),
        pltpu.VMEM_SHARED((NPAD, D), jnp.float32),
        [pltpu.SemaphoreType.DMA] * NBUF,
        pltpu.SemaphoreType.DMA,
    ],
)
def _agg_kernel(g_hbm, src_hbm, dst_hbm, zeros2_hbm, out_hbm,
                sidx, didx, rows, acc, gsems, isem):
    c = lax.axis_index("c")
    s = lax.axis_index("s")
    myc = lax.select(c == 0, C0, C1)
    cbase = lax.select(c == 0, s * C0, NS * C0 + s * C1)
    with jax.named_scope("agg_init"):
        pltpu.sync_copy(src_hbm.at[cbase], sidx.at[0])
        pltpu.sync_copy(dst_hbm.at[cbase], didx.at[0])
        pltpu.sync_copy(zeros2_hbm, acc.at[pl.ds(s * RPT, RPT), :])
        plsc.subcore_barrier()

    def chunk_body(ck, carry):
        par = lax.rem(ck, 2)
        nxt = lax.rem(ck + 1, 2)

        @pl.when(ck > 0)
        def _():
            pltpu.make_async_copy(src_hbm.at[cbase + ck], sidx.at[par],
                                  isem).wait()
            pltpu.make_async_copy(dst_hbm.at[cbase + ck], didx.at[par],
                                  isem).wait()

        @pl.when(ck < myc - 1)
        def _():
            pltpu.async_copy(src_hbm.at[cbase + ck + 1], sidx.at[nxt], isem)
            pltpu.async_copy(dst_hbm.at[cbase + ck + 1], didx.at[nxt], isem)

        for b in range(NBUF):
            pltpu.async_copy(g_hbm.at[sidx.at[par, b]], rows.at[b], gsems[b])
        for i in range(CHK):
            b = i % NBUF
            pltpu.make_async_copy(g_hbm.at[sidx.at[par, i]], rows.at[b],
                                  gsems[b]).wait()
            pltpu.sync_copy(rows.at[b], acc.at[didx.at[par, i]], add=True)
            if i + NBUF < CHK:
                pltpu.async_copy(g_hbm.at[sidx.at[par, i + NBUF]], rows.at[b],
                                 gsems[b])
        return carry

    with jax.named_scope("agg_loop"):
        lax.fori_loop(0, myc, chunk_body, 0)
        plsc.subcore_barrier()
    with jax.named_scope("agg_wb"):
        pltpu.sync_copy(acc.at[pl.ds(s * RPT, RPT), :],
                        out_hbm.at[c, pl.ds(s * RPT, RPT), :])


_BR = 400
_GRID = N // _BR

_cnt_spec = pl.BlockSpec((NC, 2, _BR, 1), lambda i: (0, 0, i, 0))
_row_spec = pl.BlockSpec((_BR, D), lambda i: (i, 0))
_w_spec = pl.BlockSpec((D, D), lambda i: (0, 0))
_b_spec = pl.BlockSpec((1, D), lambda i: (0, 0))
_p_spec = pl.BlockSpec((NC, _BR, D), lambda i: (0, i, 0))


def _dot(a, b):
    return jax.lax.dot_general(a, b, (((1,), (0,)), ((), ())),
                               precision=jax.lax.Precision.HIGHEST,
                               preferred_element_type=jnp.float32)


def _stage1_body(feat_ref, w_ref, cnt_ref, o_ref):
    ns = jax.lax.rsqrt(cnt_ref[0, 0] + cnt_ref[1, 0] + 1.0)
    o_ref[...] = _dot(feat_ref[...] * ns, w_ref[...])


_stage1 = pl.pallas_call(
    _stage1_body,
    grid=(_GRID,),
    in_specs=[_row_spec, _w_spec, _cnt_spec],
    out_specs=_row_spec,
    out_shape=jax.ShapeDtypeStruct((N, D), jnp.float32),
)


def _stage_mid_body(p_ref, g_ref, cnt_ref, b_ref, w_ref, o_ref):
    nd = jax.lax.rsqrt(cnt_ref[0, 1] + cnt_ref[1, 1] + 1.0)
    ns = jax.lax.rsqrt(cnt_ref[0, 0] + cnt_ref[1, 0] + 1.0)
    agg = p_ref[0] + p_ref[1] + g_ref[...]
    h = jnp.maximum(agg * nd + b_ref[...], 0.0)
    o_ref[...] = _dot(h * ns, w_ref[...])


_stage_mid = pl.pallas_call(
    _stage_mid_body,
    grid=(_GRID,),
    in_specs=[_p_spec, _row_spec, _cnt_spec, _b_spec, _w_spec],
    out_specs=_row_spec,
    out_shape=jax.ShapeDtypeStruct((N, D), jnp.float32),
)


def _stage_final_body(p_ref, g_ref, cnt_ref, b_ref, o_ref):
    nd = jax.lax.rsqrt(cnt_ref[0, 1] + cnt_ref[1, 1] + 1.0)
    agg = p_ref[0] + p_ref[1] + g_ref[...]
    o_ref[...] = agg * nd + b_ref[...]


_stage_final = pl.pallas_call(
    _stage_final_body,
    grid=(_GRID,),
    in_specs=[_p_spec, _row_spec, _cnt_spec, _b_spec],
    out_specs=_row_spec,
    out_shape=jax.ShapeDtypeStruct((N, D), jnp.float32),
)


def kernel(feat, edge_index, W1, b1, W2, b2, W3, b3):
    pad = EPAD - E
    src = edge_index[0]
    dst = edge_index[1]
    gpad = jnp.arange(pad, dtype=jnp.int32) * 37 % N
    src_a = jnp.concatenate([src, gpad]).reshape(NCHUNKS, CHK, K)
    trash = N + (jnp.arange(pad, dtype=jnp.int32) % (NPAD - N))
    src_d = jnp.concatenate([src, trash]).reshape(NW, NB, K)
    dst_p = jnp.concatenate([dst, trash]).reshape(NW, NB, K)
    dst_a = dst_p.reshape(NCHUNKS, CHK, K)

    zeros1 = jnp.zeros((RPT,), jnp.float32)
    zeros2 = jnp.zeros((RPT, D), jnp.float32)
    ones = jnp.ones((K,), jnp.float32)

    cnt = _deg_kernel(src_d, dst_p, zeros1, ones)
    cnt4 = cnt.reshape(NC, 2, NPAD, 1)

    b1r = b1.reshape(1, D)
    b2r = b2.reshape(1, D)
    b3r = b3.reshape(1, D)

    g1 = _stage1(feat, W1, cnt4)
    p1 = _agg_kernel(g1, src_a, dst_a, zeros2)
    g2 = _stage_mid(p1, g1, cnt4, b1r, W2)
    p2 = _agg_kernel(g2, src_a, dst_a, zeros2)
    g3 = _stage_mid(p2, g2, cnt4, b2r, W3)
    p3 = _agg_kernel(g3, src_a, dst_a, zeros2)
    return _stage_final(p3, g3, cnt4, b3r)


# speedup vs baseline: 3.7653x; 1.0847x over previous
import functools

import jax
import jax.numpy as jnp
from jax import lax
from jax.experimental import pallas as pl
from jax.experimental.pallas import tpu as pltpu
from jax.experimental.pallas import tpu_sc as plsc

N = 10000
D = 128
E = 320000

NC = 2
NS = 16
NW = NC * NS

K = 128
NBUF = 2
CHK = 10
C0 = 8
C1 = 8
NCHUNKS = NS * (C0 + C1)
NB = 80
CE = NB * K
EPAD = NCHUNKS * CHK * K

NPAD = 10240
RPT = NPAD // NS

_mesh = plsc.VectorSubcoreMesh(core_axis_name="c", subcore_axis_name="s")


@functools.partial(
    pl.kernel,
    out_type=jax.ShapeDtypeStruct((NC, 2, NPAD), jnp.float32),
    mesh=_mesh,
    scratch_types=[
        pltpu.VMEM((NB, K), jnp.int32),
        pltpu.VMEM((NB, K), jnp.int32),
        pltpu.VMEM((K,), jnp.float32),
        pltpu.VMEM_SHARED((NPAD,), jnp.float32),
        pltpu.VMEM_SHARED((NPAD,), jnp.float32),
    ],
)
def _deg_kernel(src_hbm, dst_hbm, zeros1_hbm, ones_hbm, out_hbm,
                sidx, didx, ones_v, acc_s, acc_d):
    c = lax.axis_index("c")
    s = lax.axis_index("s")
    wid = s * NC + c
    pltpu.sync_copy(ones_hbm, ones_v)
    pltpu.sync_copy(src_hbm.at[wid], sidx)
    pltpu.sync_copy(dst_hbm.at[wid], didx)
    pltpu.sync_copy(zeros1_hbm, acc_s.at[pl.ds(s * RPT, RPT)])
    pltpu.sync_copy(zeros1_hbm, acc_d.at[pl.ds(s * RPT, RPT)])
    plsc.subcore_barrier()

    def step(j, carry):
        pltpu.sync_copy(ones_v, acc_s.at[sidx.at[j]], add=True)
        pltpu.sync_copy(ones_v, acc_d.at[didx.at[j]], add=True)
        return carry

    lax.fori_loop(0, NB, step, 0)
    plsc.subcore_barrier()
    pltpu.sync_copy(acc_s.at[pl.ds(s * RPT, RPT)],
                    out_hbm.at[c, 0, pl.ds(s * RPT, RPT)])
    pltpu.sync_copy(acc_d.at[pl.ds(s * RPT, RPT)],
                    out_hbm.at[c, 1, pl.ds(s * RPT, RPT)])


@functools.partial(
    pl.kernel,
    out_type=jax.ShapeDtypeStruct((NC, NPAD, D), jnp.float32),
    mesh=_mesh,
    scratch_types=[
        pltpu.VMEM((2, CHK, K), jnp.int32),
        pltpu.VMEM((2, CHK, K), jnp.int32),
        pltpu.VMEM((NBUF, K, D), jnp.float32),
        pltpu.VMEM_SHARED((NPAD, D), jnp.float32),
        [pltpu.SemaphoreType.DMA] * NBUF,
        pltpu.SemaphoreType.DMA,
    ],
)
def _agg_kernel(g_hbm, src_hbm, dst_hbm, zeros2_hbm, out_hbm,
                sidx, didx, rows, acc, gsems, isem):
    c = lax.axis_index("c")
    s = lax.axis_index("s")
    myc = lax.select(c == 0, C0, C1)
    cbase = lax.select(c == 0, s * C0, NS * C0 + s * C1)
    pltpu.sync_copy(src_hbm.at[cbase], sidx.at[0])
    pltpu.sync_copy(dst_hbm.at[cbase], didx.at[0])
    pltpu.sync_copy(zeros2_hbm, acc.at[pl.ds(s * RPT, RPT), :])
    plsc.subcore_barrier()

    def chunk_body(ck, carry):
        par = lax.rem(ck, 2)
        nxt = lax.rem(ck + 1, 2)

        @pl.when(ck > 0)
        def _():
            pltpu.make_async_copy(src_hbm.at[cbase + ck], sidx.at[par],
                                  isem).wait()
            pltpu.make_async_copy(dst_hbm.at[cbase + ck], didx.at[par],
                                  isem).wait()

        @pl.when(ck < myc - 1)
        def _():
            pltpu.async_copy(src_hbm.at[cbase + ck + 1], sidx.at[nxt], isem)
            pltpu.async_copy(dst_hbm.at[cbase + ck + 1], didx.at[nxt], isem)

        for b in range(NBUF):
            pltpu.async_copy(g_hbm.at[sidx.at[par, b]], rows.at[b], gsems[b])
        for i in range(CHK):
            b = i % NBUF
            pltpu.make_async_copy(g_hbm.at[sidx.at[par, i]], rows.at[b],
                                  gsems[b]).wait()
            pltpu.sync_copy(rows.at[b], acc.at[didx.at[par, i]], add=True)
            if i + NBUF < CHK:
                pltpu.async_copy(g_hbm.at[sidx.at[par, i + NBUF]], rows.at[b],
                                 gsems[b])
        return carry

    lax.fori_loop(0, myc, chunk_body, 0)
    plsc.subcore_barrier()
    pltpu.sync_copy(acc.at[pl.ds(s * RPT, RPT), :],
                    out_hbm.at[c, pl.ds(s * RPT, RPT), :])


_BR = 2000
_GRID = N // _BR

_cnt_spec = pl.BlockSpec((NC, 2, _BR, 1), lambda i: (0, 0, i, 0))
_row_spec = pl.BlockSpec((_BR, D), lambda i: (i, 0))
_w_spec = pl.BlockSpec((D, D), lambda i: (0, 0))
_b_spec = pl.BlockSpec((1, D), lambda i: (0, 0))
_p_spec = pl.BlockSpec((NC, _BR, D), lambda i: (0, i, 0))


def _dot(a, b):
    return jax.lax.dot_general(a, b, (((1,), (0,)), ((), ())),
                               precision=jax.lax.Precision.HIGHEST,
                               preferred_element_type=jnp.float32)


def _stage1_body(feat_ref, w_ref, cnt_ref, o_ref):
    ns = jax.lax.rsqrt(cnt_ref[0, 0] + cnt_ref[1, 0] + 1.0)
    o_ref[...] = _dot(feat_ref[...] * ns, w_ref[...])


_stage1 = pl.pallas_call(
    _stage1_body,
    grid=(_GRID,),
    in_specs=[_row_spec, _w_spec, _cnt_spec],
    out_specs=_row_spec,
    out_shape=jax.ShapeDtypeStruct((N, D), jnp.float32),
)


def _stage_mid_body(p_ref, g_ref, cnt_ref, b_ref, w_ref, o_ref):
    nd = jax.lax.rsqrt(cnt_ref[0, 1] + cnt_ref[1, 1] + 1.0)
    ns = jax.lax.rsqrt(cnt_ref[0, 0] + cnt_ref[1, 0] + 1.0)
    agg = p_ref[0] + p_ref[1] + g_ref[...]
    h = jnp.maximum(agg * nd + b_ref[...], 0.0)
    o_ref[...] = _dot(h * ns, w_ref[...])


_stage_mid = pl.pallas_call(
    _stage_mid_body,
    grid=(_GRID,),
    in_specs=[_p_spec, _row_spec, _cnt_spec, _b_spec, _w_spec],
    out_specs=_row_spec,
    out_shape=jax.ShapeDtypeStruct((N, D), jnp.float32),
)


def _stage_final_body(p_ref, g_ref, cnt_ref, b_ref, o_ref):
    nd = jax.lax.rsqrt(cnt_ref[0, 1] + cnt_ref[1, 1] + 1.0)
    agg = p_ref[0] + p_ref[1] + g_ref[...]
    o_ref[...] = agg * nd + b_ref[...]


_stage_final = pl.pallas_call(
    _stage_final_body,
    grid=(_GRID,),
    in_specs=[_p_spec, _row_spec, _cnt_spec, _b_spec],
    out_specs=_row_spec,
    out_shape=jax.ShapeDtypeStruct((N, D), jnp.float32),
)


def kernel(feat, edge_index, W1, b1, W2, b2, W3, b3):
    pad = EPAD - E
    src = edge_index[0]
    dst = edge_index[1]
    gpad = jnp.arange(pad, dtype=jnp.int32) * 37 % N
    src_a = jnp.concatenate([src, gpad]).reshape(NCHUNKS, CHK, K)
    trash = N + (jnp.arange(pad, dtype=jnp.int32) % (NPAD - N))
    src_d = jnp.concatenate([src, trash]).reshape(NW, NB, K)
    dst_p = jnp.concatenate([dst, trash]).reshape(NW, NB, K)
    dst_a = dst_p.reshape(NCHUNKS, CHK, K)

    zeros1 = jnp.zeros((RPT,), jnp.float32)
    zeros2 = jnp.zeros((RPT, D), jnp.float32)
    ones = jnp.ones((K,), jnp.float32)

    cnt = _deg_kernel(src_d, dst_p, zeros1, ones)
    cnt4 = cnt.reshape(NC, 2, NPAD, 1)

    b1r = b1.reshape(1, D)
    b2r = b2.reshape(1, D)
    b3r = b3.reshape(1, D)

    g1 = _stage1(feat, W1, cnt4)
    p1 = _agg_kernel(g1, src_a, dst_a, zeros2)
    g2 = _stage_mid(p1, g1, cnt4, b1r, W2)
    p2 = _agg_kernel(g2, src_a, dst_a, zeros2)
    g3 = _stage_mid(p2, g2, cnt4, b2r, W3)
    p3 = _agg_kernel(g3, src_a, dst_a, zeros2)
    return _stage_final(p3, g3, cnt4, b3r)
